# baseline (speedup 1.0000x reference)
import functools
import math

import numpy as np
import jax
import jax.numpy as jnp
from jax import lax
from jax.experimental import pallas as pl
from jax.experimental.pallas import tpu as pltpu
from jax.experimental.pallas import tpu_sc as plsc

_D = 64
_SEQ = 200
_CHUNK = 128
_NBUF = 4
_LANES = 16


def _pe_doubled():
    position = np.arange(0, _SEQ)[:, None].astype(np.float32)
    div_term = np.exp(
        np.arange(0, _D, 2).astype(np.float32) * (-math.log(10000.0) / _D))
    pe = np.zeros((_SEQ, _D), dtype=np.float32)
    pe[:, 0::2] = np.sin(position * div_term)
    pe[:, 1::2] = np.cos(position * div_term)
    return jnp.asarray(np.concatenate([pe, pe], axis=0))


def _build(n_rows, nw):
    n_per_w = n_rows // nw
    n_chunks = n_per_w // _CHUNK
    n_rounds = n_chunks // _NBUF
    scale = float(math.sqrt(_D))

    mesh = plsc.VectorSubcoreMesh(core_axis_name="c", subcore_axis_name="s")

    @functools.partial(
        pl.kernel,
        mesh=mesh,
        out_type=jax.ShapeDtypeStruct((n_rows, _D), jnp.float32),
        scratch_types=[
            pltpu.VMEM((n_chunks, _CHUNK), jnp.int32),
            pltpu.VMEM((2 * _SEQ, _D), jnp.float32),
            pltpu.VMEM((_NBUF, _CHUNK, _D), jnp.float32),
            pltpu.VMEM((_NBUF, _CHUNK, _D), jnp.float32),
        ]
        + [pltpu.SemaphoreType.DMA] * (2 * _NBUF),
        compiler_params=pltpu.CompilerParams(use_tc_tiling_on_sc=False),
    )
    def run(table_h, idx_h, pe_h, out_h, idx_v, pe_v, g_v, s_v, *sems):
        gsem = sems[:_NBUF]
        ssem = sems[_NBUF:]
        ncores = plsc.get_sparse_core_info().num_cores
        wid = lax.axis_index("s") * ncores + lax.axis_index("c")
        base = wid * n_per_w

        pltpu.sync_copy(idx_h.at[wid], idx_v)
        pltpu.sync_copy(pe_h, pe_v)

        def gather_start(c, b):
            pltpu.async_copy(table_h.at[idx_v.at[c]], g_v.at[b], gsem[b])

        def gather_wait(c, b):
            pltpu.make_async_copy(
                table_h.at[idx_v.at[c]], g_v.at[b], gsem[b]).wait()

        def scatter_start(c, b):
            pltpu.async_copy(
                s_v.at[b], out_h.at[pl.ds(base + c * _CHUNK, _CHUNK)], ssem[b])

        def scatter_wait(c, b):
            pltpu.make_async_copy(
                s_v.at[b], out_h.at[pl.ds(base + c * _CHUNK, _CHUNK)],
                ssem[b]).wait()

        def compute(c, b):
            s0 = lax.rem(c * _CHUNK, _SEQ)

            def row(r, carry):
                for j in range(_D // _LANES):
                    sl = pl.ds(j * _LANES, _LANES)
                    g = g_v[b, r, sl]
                    p = pe_v[carry + r, sl]
                    s_v[b, r, sl] = g * scale + p
                return carry

            lax.fori_loop(0, _CHUNK, row, s0, unroll=8)

        def step(c, b, first, last):
            gather_wait(c, b)
            if not first:
                scatter_wait(c - _NBUF, b)
            compute(c, b)
            scatter_start(c, b)
            if not last:
                gather_start(c + _NBUF, b)

        for b in range(_NBUF):
            gather_start(b, b)

        for b in range(_NBUF):
            step(b, b, first=True, last=False)

        def round_body(k, _):
            for b in range(_NBUF):
                step(k * _NBUF + b, b, first=False, last=False)
            return 0

        lax.fori_loop(1, n_rounds - 1, round_body, 0)

        for b in range(_NBUF):
            step((n_rounds - 1) * _NBUF + b, b, first=False, last=True)

        for b in range(_NBUF):
            scatter_wait((n_rounds - 1) * _NBUF + b, b)

    return run


def kernel(x, token_embedding):
    bsz, seq = x.shape
    n_rows = bsz * seq
    info = plsc.get_sparse_core_info()
    nw = info.num_cores * info.num_subcores
    idx3 = x.reshape(-1).astype(jnp.int32).reshape(nw, n_rows // nw // _CHUNK,
                                                   _CHUNK)
    run = _build(n_rows, nw)
    out = run(token_embedding, idx3, _pe_doubled())
    return out.reshape(bsz, seq, _D)
